# baseline (speedup 1.0000x reference)
import functools

import jax
import jax.numpy as jnp
from jax import lax
from jax.experimental import pallas as pl
from jax.experimental.pallas import tpu as pltpu
from jax.experimental.pallas import tpu_sc as plsc

NEG_SLOPE = 0.2
EPS = 1e-9

NC = 2
NS = 16
LANES = 16

BM = 1024


def _vgather(x, idx):
    dn = lax.GatherDimensionNumbers(offset_dims=(), collapsed_slice_dims=(0,),
                                    start_index_map=(0,))
    return lax.gather(x, idx[:, None], dn, (1,),
                      mode=lax.GatherScatterMode.PROMISE_IN_BOUNDS)


def _tc_stage(x, pn, W, al, ar, *, npad, d):
    first = x is not None
    grid = npad // BM

    def body(*refs):
        if first:
            x_ref, w_ref, al_ref, ar_ref, feat_ref, el_ref, er_ref = refs
            xb = x_ref[...]
        else:
            pn_ref, w_ref, al_ref, ar_ref, feat_ref, el_ref, er_ref = refs
            xb = pn_ref[0] + pn_ref[1]
        f = jnp.dot(xb, w_ref[...], preferred_element_type=jnp.float32)
        feat_ref[...] = f
        el = jnp.sum(f * al_ref[...], axis=1)
        er = jnp.sum(f * ar_ref[...], axis=1)
        el_ref[...] = jnp.broadcast_to(el[None, :], (8, BM))
        er_ref[...] = jnp.broadcast_to(er[None, :], (8, BM))

    if first:
        data_specs = [pl.BlockSpec((BM, d), lambda i: (i, 0))]
        data_args = (x,)
    else:
        data_specs = [pl.BlockSpec((NC, BM, d), lambda i: (0, i, 0))]
        data_args = (pn,)

    return pl.pallas_call(
        body,
        grid=(grid,),
        in_specs=data_specs + [
            pl.BlockSpec((d, d), lambda i: (0, 0)),
            pl.BlockSpec((1, d), lambda i: (0, 0)),
            pl.BlockSpec((1, d), lambda i: (0, 0)),
        ],
        out_specs=[
            pl.BlockSpec((BM, d), lambda i: (i, 0)),
            pl.BlockSpec((8, BM), lambda i: (0, i)),
            pl.BlockSpec((8, BM), lambda i: (0, i)),
        ],
        out_shape=[
            jax.ShapeDtypeStruct((npad, d), jnp.float32),
            jax.ShapeDtypeStruct((8, npad), jnp.float32),
            jax.ShapeDtypeStruct((8, npad), jnp.float32),
        ],
    )(*data_args, W, al, ar)


def _tc_merge(pn, *, npad, d):
    grid = npad // BM

    def body(pn_ref, out_ref):
        out_ref[...] = pn_ref[0] + pn_ref[1]

    return pl.pallas_call(
        body,
        grid=(grid,),
        in_specs=[pl.BlockSpec((NC, BM, d), lambda i: (0, i, 0))],
        out_specs=pl.BlockSpec((BM, d), lambda i: (i, 0)),
        out_shape=jax.ShapeDtypeStruct((npad, d), jnp.float32),
    )(pn)


@functools.lru_cache(maxsize=None)
def _sc_aggregate_kernel(npad, d, nchunk, e_real, sp):
    eptile = nchunk * 128
    rpt = npad // NS
    nslab = rpt // 64
    mesh = plsc.VectorSubcoreMesh(core_axis_name="c", subcore_axis_name="s")

    @functools.partial(
        pl.kernel,
        out_type=jax.ShapeDtypeStruct((NC, npad, d), jnp.float32),
        mesh=mesh,
        compiler_params=pltpu.CompilerParams(needs_layout_passes=False),
        scratch_types=(
            pltpu.VMEM((npad,), jnp.float32),
            pltpu.VMEM((npad,), jnp.float32),
            pltpu.VMEM((64, d), jnp.float32),
            pltpu.VMEM((64, d), jnp.float32),
            pltpu.VMEM((8, 128), jnp.int32),
            pltpu.VMEM((8, 128), jnp.int32),
            pltpu.VMEM((128,), jnp.int32),
            pltpu.VMEM((128,), jnp.float32),
            pltpu.VMEM((8, 128), jnp.int32),
            pltpu.VMEM((8, 128), jnp.float32),
            pltpu.VMEM((rpt,), jnp.float32),
            pltpu.VMEM_SHARED((npad, d), jnp.float32),
            pltpu.VMEM_SHARED((npad,), jnp.float32),
            pltpu.SemaphoreType.DMA,
            pltpu.SemaphoreType.DMA,
            pltpu.SemaphoreType.DMA,
        ),
    )
    def k(feat_hbm, el8_hbm, er8_hbm, src_hbm, dst_hbm, outn_hbm,
          el_v, er_v, rows_g, rows_s, srcst, dstst, ddst_v, dval_v,
          ddst8, dval8, den_v, accn, accd, sem_g, sem_s, sem_d):
        c = lax.axis_index("c")
        s = lax.axis_index("s")
        iota16 = lax.iota(jnp.int32, LANES)
        zeros16 = jnp.zeros((LANES,), jnp.float32)
        base = s * rpt

        pltpu.sync_copy(el8_hbm.at[0], el_v)
        pltpu.sync_copy(er8_hbm.at[0], er_v)

        def zrow(r, carry):
            for kk in range(d // LANES):
                rows_g[r, pl.ds(kk * LANES, LANES)] = zeros16
            return carry
        lax.fori_loop(0, 64, zrow, 0)

        def zden(r, carry):
            den_v[pl.ds(r * LANES, LANES)] = zeros16
            return carry
        lax.fori_loop(0, rpt // LANES, zden, 0)

        for t in range(nslab):
            pltpu.sync_copy(rows_g, accn.at[pl.ds(base + t * 64, 64)])
        pltpu.sync_copy(den_v, accd.at[pl.ds(base, rpt)])
        plsc.subcore_barrier()

        lo1 = jnp.where(c == 0, sp, 0)
        ng1 = jnp.where(c == 0, (nchunk - sp) // 8, sp // 8)
        lo2 = jnp.where(c == 0, 0, sp)
        ng2 = jnp.where(c == 0, sp // 8, (nchunk - sp) // 8)

        def wonly16(q, ch, off, j):
            s16 = srcst[q, pl.ds(off + j * LANES, LANES)]
            d16 = dstst[q, pl.ds(off + j * LANES, LANES)]
            ev = plsc.load_gather(el_v, [s16]) + plsc.load_gather(er_v, [d16])
            ev = jnp.where(ev >= 0.0, ev, NEG_SLOPE * ev)
            wv = jnp.exp(ev)
            gid = s * eptile + ch * 128 + off + j * LANES + iota16
            wv = jnp.where(gid < e_real, wv, 0.0)
            return d16, wv

        def wgroup16(q, ch, off, j):
            d16, wv = wonly16(q, ch, off, j)
            ddst_v[pl.ds(off + j * LANES, LANES)] = d16
            dval_v[pl.ds(off + j * LANES, LANES)] = wv
            return wv

        def grp1(g, carry):
            glo = lo1 + g * 8
            pltpu.sync_copy(src_hbm.at[s, pl.ds(glo, 8)], srcst)
            pltpu.sync_copy(dst_hbm.at[s, pl.ds(glo, 8)], dstst)
            for q in range(8):
                ch = glo + q
                for j in range(8):
                    d16, wv = wonly16(q, ch, 0, j)
                    ddst8[q, pl.ds(j * LANES, LANES)] = d16
                    dval8[q, pl.ds(j * LANES, LANES)] = wv
                pltpu.async_copy(dval8.at[q], accd.at[ddst8.at[q]], sem_d, add=True)
            for q in range(8):
                pltpu.make_async_copy(
                    dval8.at[q], accd.at[ddst8.at[q]], sem_d).wait()
            return carry
        lax.fori_loop(0, ng1, grp1, 0)

        def grp2(g, carry):
            glo = lo2 + g * 8
            pltpu.sync_copy(src_hbm.at[s, pl.ds(glo, 8)], srcst)
            pltpu.sync_copy(dst_hbm.at[s, pl.ds(glo, 8)], dstst)

            def scale64(wvs):
                def scale(jj, carry3, _wvs=wvs):
                    wv = _wvs[0]
                    for jw in range(1, 4):
                        wv = jnp.where(jj == jw, _wvs[jw], wv)
                    for r in range(LANES):
                        row = jj * LANES + r
                        a = _vgather(wv, jnp.full((LANES,), r, jnp.int32))
                        for kk in range(d // LANES):
                            rows_s[row, pl.ds(kk * LANES, LANES)] = (
                                rows_g[row, pl.ds(kk * LANES, LANES)] * a)
                    return carry3
                lax.fori_loop(0, 4, scale, 0)

            def ch2(q, carry2):
                ch = glo + q
                pltpu.async_copy(
                    feat_hbm.at[srcst.at[q, pl.ds(0, 64)]], rows_g, sem_g)
                wvs_a = [wgroup16(q, ch, 0, j) for j in range(4)]
                pltpu.make_async_copy(
                    feat_hbm.at[srcst.at[q, pl.ds(0, 64)]], rows_g,
                    sem_g).wait()
                scale64(wvs_a)
                pltpu.async_copy(
                    rows_s, accn.at[dstst.at[q, pl.ds(0, 64)]], sem_s,
                    add=True)
                pltpu.async_copy(
                    feat_hbm.at[srcst.at[q, pl.ds(64, 64)]], rows_g, sem_g)
                wvs_b = [wgroup16(q, ch, 64, j) for j in range(4)]
                pltpu.async_copy(dval_v, accd.at[ddst_v], sem_d, add=True)
                pltpu.make_async_copy(
                    feat_hbm.at[srcst.at[q, pl.ds(64, 64)]], rows_g,
                    sem_g).wait()
                pltpu.make_async_copy(
                    rows_s, accn.at[dstst.at[q, pl.ds(0, 64)]], sem_s).wait()
                scale64(wvs_b)
                pltpu.async_copy(
                    rows_s, accn.at[dstst.at[q, pl.ds(64, 64)]], sem_s,
                    add=True)
                pltpu.make_async_copy(dval_v, accd.at[ddst_v], sem_d).wait()
                pltpu.make_async_copy(
                    rows_s, accn.at[dstst.at[q, pl.ds(64, 64)]], sem_s).wait()
                return carry2
            lax.fori_loop(0, 8, ch2, 0)
            return carry
        lax.fori_loop(0, ng2, grp2, 0)
        plsc.subcore_barrier()

        pltpu.sync_copy(accd.at[pl.ds(base, rpt)], den_v)
        for t in range(nslab):
            pltpu.sync_copy(accn.at[pl.ds(base + t * 64, 64)], rows_g)

            def divg(jj, carry, _t=t):
                r0 = jj * LANES
                dv = den_v[pl.ds(_t * 64 + r0, LANES)]
                inv = 1.0 / (dv + EPS)
                for r in range(LANES):
                    row = r0 + r
                    a = _vgather(inv, jnp.full((LANES,), r, jnp.int32))
                    for kk in range(d // LANES):
                        rows_g[row, pl.ds(kk * LANES, LANES)] = (
                            rows_g[row, pl.ds(kk * LANES, LANES)] * a)
                return carry
            lax.fori_loop(0, 4, divg, 0)
            pltpu.sync_copy(rows_g, outn_hbm.at[c, pl.ds(base + t * 64, 64)])

    return k


SPLIT_FRAC = 0.80


def _sc_aggregate(feat, el8, er8, src3, dst3, *, npad, d, nchunk, e_real):
    sp = max(8, int(nchunk * SPLIT_FRAC) // 8 * 8)
    return _sc_aggregate_kernel(npad, d, nchunk, e_real, sp)(
        feat, el8, er8, src3, dst3)


def kernel(inputs, g, W1, al1, ar1, W2, al2, ar2):
    n, f = inputs.shape
    d = W1.shape[1]
    e = g.shape[1]

    npad = -(-n // 2048) * 2048
    nchunk = -(-(-(-e // (NS * 128))) // 16) * 16
    epad = NS * nchunk * 128

    x = jnp.zeros((npad, f), jnp.float32).at[:n, :].set(inputs)
    src = g[0].astype(jnp.int32)
    dst = g[1].astype(jnp.int32)
    src3 = jnp.zeros((epad,), jnp.int32).at[:e].set(src).reshape(NS, nchunk, 128)
    dst3 = jnp.zeros((epad,), jnp.int32).at[:e].set(dst).reshape(NS, nchunk, 128)

    feat1, el81, er81 = _tc_stage(x, None, W1, al1, ar1, npad=npad, d=d)
    pn1 = _sc_aggregate(feat1, el81, er81, src3, dst3,
                        npad=npad, d=d, nchunk=nchunk, e_real=e)
    feat2, el82, er82 = _tc_stage(None, pn1, W2, al2, ar2, npad=npad, d=d)
    pn2 = _sc_aggregate(feat2, el82, er82, src3, dst3,
                        npad=npad, d=d, nchunk=nchunk, e_real=e)
    out = _tc_merge(pn2, npad=npad, d=d)
    return out[:n]


# speedup vs baseline: 1.0811x; 1.0811x over previous
import functools

import jax
import jax.numpy as jnp
from jax import lax
from jax.experimental import pallas as pl
from jax.experimental.pallas import tpu as pltpu
from jax.experimental.pallas import tpu_sc as plsc

NEG_SLOPE = 0.2
EPS = 1e-9

NC = 2
NS = 16
LANES = 16

BM = 1024


def _vgather(x, idx):
    dn = lax.GatherDimensionNumbers(offset_dims=(), collapsed_slice_dims=(0,),
                                    start_index_map=(0,))
    return lax.gather(x, idx[:, None], dn, (1,),
                      mode=lax.GatherScatterMode.PROMISE_IN_BOUNDS)


def _tc_stage(x, pn, W, al, ar, *, npad, d):
    first = x is not None
    grid = npad // BM

    def body(*refs):
        if first:
            x_ref, w_ref, al_ref, ar_ref, feat_ref, el_ref, er_ref = refs
            xb = x_ref[...]
        else:
            pn_ref, w_ref, al_ref, ar_ref, feat_ref, el_ref, er_ref = refs
            xb = pn_ref[0] + pn_ref[1]
        f = jnp.dot(xb, w_ref[...], preferred_element_type=jnp.float32)
        feat_ref[...] = f
        el = jnp.sum(f * al_ref[...], axis=1)
        er = jnp.sum(f * ar_ref[...], axis=1)
        el_ref[...] = jnp.broadcast_to(el[None, :], (8, BM))
        er_ref[...] = jnp.broadcast_to(er[None, :], (8, BM))

    if first:
        data_specs = [pl.BlockSpec((BM, d), lambda i: (i, 0))]
        data_args = (x,)
    else:
        data_specs = [pl.BlockSpec((NC, BM, d), lambda i: (0, i, 0))]
        data_args = (pn,)

    return pl.pallas_call(
        body,
        grid=(grid,),
        in_specs=data_specs + [
            pl.BlockSpec((d, d), lambda i: (0, 0)),
            pl.BlockSpec((1, d), lambda i: (0, 0)),
            pl.BlockSpec((1, d), lambda i: (0, 0)),
        ],
        out_specs=[
            pl.BlockSpec((BM, d), lambda i: (i, 0)),
            pl.BlockSpec((8, BM), lambda i: (0, i)),
            pl.BlockSpec((8, BM), lambda i: (0, i)),
        ],
        out_shape=[
            jax.ShapeDtypeStruct((npad, d), jnp.float32),
            jax.ShapeDtypeStruct((8, npad), jnp.float32),
            jax.ShapeDtypeStruct((8, npad), jnp.float32),
        ],
    )(*data_args, W, al, ar)


def _tc_merge(pn, *, npad, d):
    grid = npad // BM

    def body(pn_ref, out_ref):
        out_ref[...] = pn_ref[0] + pn_ref[1]

    return pl.pallas_call(
        body,
        grid=(grid,),
        in_specs=[pl.BlockSpec((NC, BM, d), lambda i: (0, i, 0))],
        out_specs=pl.BlockSpec((BM, d), lambda i: (i, 0)),
        out_shape=jax.ShapeDtypeStruct((npad, d), jnp.float32),
    )(pn)


@functools.lru_cache(maxsize=None)
def _sc_aggregate_kernel(npad, d, nchunk, e_real, sp):
    eptile = nchunk * 128
    rpt = npad // NS
    nslab = rpt // 64
    mesh = plsc.VectorSubcoreMesh(core_axis_name="c", subcore_axis_name="s")

    @functools.partial(
        pl.kernel,
        out_type=jax.ShapeDtypeStruct((NC, npad, d), jnp.float32),
        mesh=mesh,
        compiler_params=pltpu.CompilerParams(needs_layout_passes=False),
        scratch_types=(
            pltpu.VMEM((npad,), jnp.float32),
            pltpu.VMEM((npad,), jnp.float32),
            pltpu.VMEM((64, d), jnp.float32),
            pltpu.VMEM((64, d), jnp.float32),
            pltpu.VMEM((8, 128), jnp.int32),
            pltpu.VMEM((8, 128), jnp.int32),
            pltpu.VMEM((128,), jnp.int32),
            pltpu.VMEM((128,), jnp.float32),
            pltpu.VMEM((8, 128), jnp.int32),
            pltpu.VMEM((8, 128), jnp.float32),
            pltpu.VMEM((rpt,), jnp.float32),
            pltpu.VMEM_SHARED((npad, d), jnp.float32),
            pltpu.VMEM_SHARED((npad,), jnp.float32),
            pltpu.SemaphoreType.DMA,
            pltpu.SemaphoreType.DMA,
            pltpu.SemaphoreType.DMA,
        ),
    )
    def k(feat_hbm, el8_hbm, er8_hbm, src_hbm, dst_hbm, outn_hbm,
          el_v, er_v, rows_g, rows_s, srcst, dstst, ddst_v, dval_v,
          ddst8, dval8, den_v, accn, accd, sem_g, sem_s, sem_d):
        c = lax.axis_index("c")
        s = lax.axis_index("s")
        iota16 = lax.iota(jnp.int32, LANES)
        zeros16 = jnp.zeros((LANES,), jnp.float32)
        base = s * rpt

        pltpu.sync_copy(el8_hbm.at[0], el_v)
        pltpu.sync_copy(er8_hbm.at[0], er_v)

        def zrow(r, carry):
            for kk in range(d // LANES):
                rows_g[r, pl.ds(kk * LANES, LANES)] = zeros16
            return carry
        lax.fori_loop(0, 64, zrow, 0)

        def zden(r, carry):
            den_v[pl.ds(r * LANES, LANES)] = zeros16
            return carry
        lax.fori_loop(0, rpt // LANES, zden, 0)

        for t in range(nslab):
            pltpu.sync_copy(rows_g, accn.at[pl.ds(base + t * 64, 64)])
        pltpu.sync_copy(den_v, accd.at[pl.ds(base, rpt)])
        plsc.subcore_barrier()

        lo1 = jnp.where(c == 0, sp, 0)
        ng1 = jnp.where(c == 0, (nchunk - sp) // 8, sp // 8)
        lo2 = jnp.where(c == 0, 0, sp)
        ng2 = jnp.where(c == 0, sp // 8, (nchunk - sp) // 8)

        def wonly16(q, ch, off, j):
            s16 = srcst[q, pl.ds(off + j * LANES, LANES)]
            d16 = dstst[q, pl.ds(off + j * LANES, LANES)]
            ev = plsc.load_gather(el_v, [s16]) + plsc.load_gather(er_v, [d16])
            ev = jnp.where(ev >= 0.0, ev, NEG_SLOPE * ev)
            wv = jnp.exp(ev)
            gid = s * eptile + ch * 128 + off + j * LANES + iota16
            wv = jnp.where(gid < e_real, wv, 0.0)
            return d16, wv

        def wgroup16(q, ch, off, j):
            d16, wv = wonly16(q, ch, off, j)
            ddst_v[pl.ds(off + j * LANES, LANES)] = d16
            dval_v[pl.ds(off + j * LANES, LANES)] = wv
            return wv

        def grp1(g, carry):
            glo = lo1 + g * 8
            pltpu.sync_copy(src_hbm.at[s, pl.ds(glo, 8)], srcst)
            pltpu.sync_copy(dst_hbm.at[s, pl.ds(glo, 8)], dstst)
            for q in range(8):
                ch = glo + q
                for j in range(8):
                    d16, wv = wonly16(q, ch, 0, j)
                    ddst8[q, pl.ds(j * LANES, LANES)] = d16
                    dval8[q, pl.ds(j * LANES, LANES)] = wv
                pltpu.async_copy(dval8.at[q], accd.at[ddst8.at[q]], sem_d, add=True)
            for q in range(8):
                pltpu.make_async_copy(
                    dval8.at[q], accd.at[ddst8.at[q]], sem_d).wait()
            return carry
        lax.fori_loop(0, ng1, grp1, 0)

        def grp2(g, carry):
            glo = lo2 + g * 8
            pltpu.sync_copy(src_hbm.at[s, pl.ds(glo, 8)], srcst)
            pltpu.sync_copy(dst_hbm.at[s, pl.ds(glo, 8)], dstst)

            def scale64(wvs):
                def scale(jj, carry3, _wvs=wvs):
                    wv = _wvs[0]
                    for jw in range(1, 4):
                        wv = jnp.where(jj == jw, _wvs[jw], wv)
                    for r in range(LANES):
                        row = jj * LANES + r
                        a = _vgather(wv, jnp.full((LANES,), r, jnp.int32))
                        for kk in range(d // LANES):
                            rows_s[row, pl.ds(kk * LANES, LANES)] = (
                                rows_g[row, pl.ds(kk * LANES, LANES)] * a)
                    return carry3
                lax.fori_loop(0, 4, scale, 0)

            def ch2(q, carry2):
                ch = glo + q
                pltpu.async_copy(
                    feat_hbm.at[srcst.at[q, pl.ds(0, 64)]], rows_g, sem_g)
                wvs_a = [wgroup16(q, ch, 0, j) for j in range(4)]
                pltpu.make_async_copy(
                    feat_hbm.at[srcst.at[q, pl.ds(0, 64)]], rows_g,
                    sem_g).wait()
                scale64(wvs_a)
                pltpu.async_copy(
                    rows_s, accn.at[dstst.at[q, pl.ds(0, 64)]], sem_s,
                    add=True)
                pltpu.async_copy(
                    feat_hbm.at[srcst.at[q, pl.ds(64, 64)]], rows_g, sem_g)
                wvs_b = [wgroup16(q, ch, 64, j) for j in range(4)]
                pltpu.async_copy(dval_v, accd.at[ddst_v], sem_d, add=True)
                pltpu.make_async_copy(
                    feat_hbm.at[srcst.at[q, pl.ds(64, 64)]], rows_g,
                    sem_g).wait()
                pltpu.make_async_copy(
                    rows_s, accn.at[dstst.at[q, pl.ds(0, 64)]], sem_s).wait()
                scale64(wvs_b)
                pltpu.async_copy(
                    rows_s, accn.at[dstst.at[q, pl.ds(64, 64)]], sem_s,
                    add=True)
                pltpu.make_async_copy(dval_v, accd.at[ddst_v], sem_d).wait()
                pltpu.make_async_copy(
                    rows_s, accn.at[dstst.at[q, pl.ds(64, 64)]], sem_s).wait()
                return carry2
            lax.fori_loop(0, 8, ch2, 0)
            return carry
        lax.fori_loop(0, ng2, grp2, 0)
        plsc.subcore_barrier()

        pltpu.sync_copy(accd.at[pl.ds(base, rpt)], den_v)
        for t in range(nslab):
            pltpu.sync_copy(accn.at[pl.ds(base + t * 64, 64)], rows_g)

            def divg(jj, carry, _t=t):
                r0 = jj * LANES
                dv = den_v[pl.ds(_t * 64 + r0, LANES)]
                inv = 1.0 / (dv + EPS)
                for r in range(LANES):
                    row = r0 + r
                    a = _vgather(inv, jnp.full((LANES,), r, jnp.int32))
                    for kk in range(d // LANES):
                        rows_g[row, pl.ds(kk * LANES, LANES)] = (
                            rows_g[row, pl.ds(kk * LANES, LANES)] * a)
                return carry
            lax.fori_loop(0, 4, divg, 0)
            pltpu.sync_copy(rows_g, outn_hbm.at[c, pl.ds(base + t * 64, 64)])

    return k


SPLIT_FRAC = 0.75


def _sc_aggregate(feat, el8, er8, src3, dst3, *, npad, d, nchunk, e_real):
    sp = max(8, int(nchunk * SPLIT_FRAC) // 8 * 8)
    return _sc_aggregate_kernel(npad, d, nchunk, e_real, sp)(
        feat, el8, er8, src3, dst3)


def kernel(inputs, g, W1, al1, ar1, W2, al2, ar2):
    n, f = inputs.shape
    d = W1.shape[1]
    e = g.shape[1]

    npad = -(-n // 2048) * 2048
    nchunk = -(-(-(-e // (NS * 128))) // 16) * 16
    epad = NS * nchunk * 128

    x = jnp.zeros((npad, f), jnp.float32).at[:n, :].set(inputs)
    src = g[0].astype(jnp.int32)
    dst = g[1].astype(jnp.int32)
    src3 = jnp.zeros((epad,), jnp.int32).at[:e].set(src).reshape(NS, nchunk, 128)
    dst3 = jnp.zeros((epad,), jnp.int32).at[:e].set(dst).reshape(NS, nchunk, 128)

    feat1, el81, er81 = _tc_stage(x, None, W1, al1, ar1, npad=npad, d=d)
    pn1 = _sc_aggregate(feat1, el81, er81, src3, dst3,
                        npad=npad, d=d, nchunk=nchunk, e_real=e)
    feat2, el82, er82 = _tc_stage(None, pn1, W2, al2, ar2, npad=npad, d=d)
    pn2 = _sc_aggregate(feat2, el82, er82, src3, dst3,
                        npad=npad, d=d, nchunk=nchunk, e_real=e)
    out = _tc_merge(pn2, npad=npad, d=d)
    return out[:n]


# speedup vs baseline: 1.1537x; 1.0672x over previous
import functools

import jax
import jax.numpy as jnp
from jax import lax
from jax.experimental import pallas as pl
from jax.experimental.pallas import tpu as pltpu
from jax.experimental.pallas import tpu_sc as plsc

NEG_SLOPE = 0.2
EPS = 1e-9

NC = 2
NS = 16
LANES = 16

BM = 1024


def _vgather(x, idx):
    dn = lax.GatherDimensionNumbers(offset_dims=(), collapsed_slice_dims=(0,),
                                    start_index_map=(0,))
    return lax.gather(x, idx[:, None], dn, (1,),
                      mode=lax.GatherScatterMode.PROMISE_IN_BOUNDS)


def _tc_stage(x, pn, W, al, ar, *, npad, d):
    first = x is not None
    grid = npad // BM

    def body(*refs):
        if first:
            x_ref, w_ref, al_ref, ar_ref, feat_ref, el_ref, er_ref = refs
            xb = x_ref[...]
        else:
            pn_ref, w_ref, al_ref, ar_ref, feat_ref, el_ref, er_ref = refs
            xb = pn_ref[0] + pn_ref[1]
        f = jnp.dot(xb, w_ref[...], preferred_element_type=jnp.float32)
        feat_ref[...] = f
        el = jnp.sum(f * al_ref[...], axis=1)
        er = jnp.sum(f * ar_ref[...], axis=1)
        el_ref[...] = jnp.broadcast_to(el[None, :], (8, BM))
        er_ref[...] = jnp.broadcast_to(er[None, :], (8, BM))

    if first:
        data_specs = [pl.BlockSpec((BM, d), lambda i: (i, 0))]
        data_args = (x,)
    else:
        data_specs = [pl.BlockSpec((NC, BM, d), lambda i: (0, i, 0))]
        data_args = (pn,)

    return pl.pallas_call(
        body,
        grid=(grid,),
        in_specs=data_specs + [
            pl.BlockSpec((d, d), lambda i: (0, 0)),
            pl.BlockSpec((1, d), lambda i: (0, 0)),
            pl.BlockSpec((1, d), lambda i: (0, 0)),
        ],
        out_specs=[
            pl.BlockSpec((BM, d), lambda i: (i, 0)),
            pl.BlockSpec((8, BM), lambda i: (0, i)),
            pl.BlockSpec((8, BM), lambda i: (0, i)),
        ],
        out_shape=[
            jax.ShapeDtypeStruct((npad, d), jnp.float32),
            jax.ShapeDtypeStruct((8, npad), jnp.float32),
            jax.ShapeDtypeStruct((8, npad), jnp.float32),
        ],
    )(*data_args, W, al, ar)


def _tc_merge(pn, *, npad, d):
    grid = npad // BM

    def body(pn_ref, out_ref):
        out_ref[...] = pn_ref[0] + pn_ref[1]

    return pl.pallas_call(
        body,
        grid=(grid,),
        in_specs=[pl.BlockSpec((NC, BM, d), lambda i: (0, i, 0))],
        out_specs=pl.BlockSpec((BM, d), lambda i: (i, 0)),
        out_shape=jax.ShapeDtypeStruct((npad, d), jnp.float32),
    )(pn)


@functools.lru_cache(maxsize=None)
def _sc_aggregate_kernel(npad, d, nchunk, e_real, sp):
    eptile = nchunk * 128
    rpt = npad // NS
    nslab = rpt // 64
    mesh = plsc.VectorSubcoreMesh(core_axis_name="c", subcore_axis_name="s")

    @functools.partial(
        pl.kernel,
        out_type=jax.ShapeDtypeStruct((NC, npad, d), jnp.float32),
        mesh=mesh,
        compiler_params=pltpu.CompilerParams(needs_layout_passes=False),
        scratch_types=(
            pltpu.VMEM((npad,), jnp.float32),
            pltpu.VMEM((npad,), jnp.float32),
            pltpu.VMEM((64, d), jnp.float32),
            pltpu.VMEM((64, d), jnp.float32),
            pltpu.VMEM((8, 128), jnp.int32),
            pltpu.VMEM((8, 128), jnp.int32),
            pltpu.VMEM((128,), jnp.int32),
            pltpu.VMEM((128,), jnp.float32),
            pltpu.VMEM((8, 128), jnp.int32),
            pltpu.VMEM((8, 128), jnp.float32),
            pltpu.VMEM((rpt,), jnp.float32),
            pltpu.VMEM_SHARED((npad, d), jnp.float32),
            pltpu.VMEM_SHARED((npad,), jnp.float32),
            pltpu.SemaphoreType.DMA,
            pltpu.SemaphoreType.DMA,
            pltpu.SemaphoreType.DMA,
        ),
    )
    def k(feat_hbm, el8_hbm, er8_hbm, src_hbm, dst_hbm, outn_hbm,
          el_v, er_v, rows_g, rows_s, srcst, dstst, ddst_v, dval_v,
          ddst8, dval8, den_v, accn, accd, sem_g, sem_s, sem_d):
        c = lax.axis_index("c")
        s = lax.axis_index("s")
        iota16 = lax.iota(jnp.int32, LANES)
        zeros16 = jnp.zeros((LANES,), jnp.float32)
        base = s * rpt

        pltpu.sync_copy(el8_hbm.at[0], el_v)
        pltpu.sync_copy(er8_hbm.at[0], er_v)

        def zrow(r, carry):
            for kk in range(d // LANES):
                rows_g[r, pl.ds(kk * LANES, LANES)] = zeros16
            return carry
        lax.fori_loop(0, 64, zrow, 0)

        def zden(r, carry):
            den_v[pl.ds(r * LANES, LANES)] = zeros16
            return carry
        lax.fori_loop(0, rpt // LANES, zden, 0)

        for t in range(nslab):
            pltpu.sync_copy(rows_g, accn.at[pl.ds(base + t * 64, 64)])
        pltpu.sync_copy(den_v, accd.at[pl.ds(base, rpt)])
        plsc.subcore_barrier()

        lo1 = jnp.where(c == 0, sp, 0)
        ng1 = jnp.where(c == 0, (nchunk - sp) // 8, sp // 8)
        lo2 = jnp.where(c == 0, 0, sp)
        ng2 = jnp.where(c == 0, sp // 8, (nchunk - sp) // 8)

        def wonly16(q, ch, off, j):
            s16 = srcst[q, pl.ds(off + j * LANES, LANES)]
            d16 = dstst[q, pl.ds(off + j * LANES, LANES)]
            ev = plsc.load_gather(el_v, [s16]) + plsc.load_gather(er_v, [d16])
            ev = jnp.where(ev >= 0.0, ev, NEG_SLOPE * ev)
            wv = jnp.exp(ev)
            gid = s * eptile + ch * 128 + off + j * LANES + iota16
            wv = jnp.where(gid < e_real, wv, 0.0)
            return d16, wv

        def wgroup16(q, ch, off, j):
            d16, wv = wonly16(q, ch, off, j)
            ddst_v[pl.ds(off + j * LANES, LANES)] = d16
            dval_v[pl.ds(off + j * LANES, LANES)] = wv
            return wv

        def grp1(g, carry):
            glo = lo1 + g * 8
            pltpu.sync_copy(src_hbm.at[s, pl.ds(glo, 8)], srcst)
            pltpu.sync_copy(dst_hbm.at[s, pl.ds(glo, 8)], dstst)
            for q in range(8):
                ch = glo + q
                for j in range(8):
                    d16, wv = wonly16(q, ch, 0, j)
                    ddst8[q, pl.ds(j * LANES, LANES)] = d16
                    dval8[q, pl.ds(j * LANES, LANES)] = wv
                pltpu.async_copy(dval8.at[q], accd.at[ddst8.at[q]], sem_d, add=True)
            for q in range(8):
                pltpu.make_async_copy(
                    dval8.at[q], accd.at[ddst8.at[q]], sem_d).wait()
            return carry
        lax.fori_loop(0, ng1, grp1, 0)

        def grp2(g, carry):
            glo = lo2 + g * 8
            pltpu.sync_copy(src_hbm.at[s, pl.ds(glo, 8)], srcst)
            pltpu.sync_copy(dst_hbm.at[s, pl.ds(glo, 8)], dstst)

            def scale64(wvs):
                def scale(jj, carry3, _wvs=wvs):
                    wv = _wvs[0]
                    for jw in range(1, 4):
                        wv = jnp.where(jj == jw, _wvs[jw], wv)
                    for r in range(LANES):
                        row = jj * LANES + r
                        a = _vgather(wv, jnp.full((LANES,), r, jnp.int32))
                        for kk in range(d // LANES):
                            rows_s[row, pl.ds(kk * LANES, LANES)] = (
                                rows_g[row, pl.ds(kk * LANES, LANES)] * a)
                    return carry3
                lax.fori_loop(0, 4, scale, 0)

            def ch2(q, carry2):
                ch = glo + q
                pltpu.async_copy(
                    feat_hbm.at[srcst.at[q, pl.ds(0, 64)]], rows_g, sem_g)
                wvs_a = [wgroup16(q, ch, 0, j) for j in range(4)]
                pltpu.make_async_copy(
                    feat_hbm.at[srcst.at[q, pl.ds(0, 64)]], rows_g,
                    sem_g).wait()
                scale64(wvs_a)
                pltpu.async_copy(
                    rows_s, accn.at[dstst.at[q, pl.ds(0, 64)]], sem_s,
                    add=True)
                pltpu.async_copy(
                    feat_hbm.at[srcst.at[q, pl.ds(64, 64)]], rows_g, sem_g)
                wvs_b = [wgroup16(q, ch, 64, j) for j in range(4)]
                pltpu.async_copy(dval_v, accd.at[ddst_v], sem_d, add=True)
                pltpu.make_async_copy(
                    feat_hbm.at[srcst.at[q, pl.ds(64, 64)]], rows_g,
                    sem_g).wait()
                pltpu.make_async_copy(
                    rows_s, accn.at[dstst.at[q, pl.ds(0, 64)]], sem_s).wait()
                scale64(wvs_b)
                pltpu.async_copy(
                    rows_s, accn.at[dstst.at[q, pl.ds(64, 64)]], sem_s,
                    add=True)
                pltpu.make_async_copy(dval_v, accd.at[ddst_v], sem_d).wait()
                pltpu.make_async_copy(
                    rows_s, accn.at[dstst.at[q, pl.ds(64, 64)]], sem_s).wait()
                return carry2
            lax.fori_loop(0, 8, ch2, 0)
            return carry
        lax.fori_loop(0, ng2, grp2, 0)
        plsc.subcore_barrier()

        pltpu.sync_copy(accd.at[pl.ds(base, rpt)], den_v)
        for t in range(nslab):
            pltpu.sync_copy(accn.at[pl.ds(base + t * 64, 64)], rows_g)

            def divg(jj, carry, _t=t):
                r0 = jj * LANES
                dv = den_v[pl.ds(_t * 64 + r0, LANES)]
                inv = 1.0 / (dv + EPS)
                for r in range(LANES):
                    row = r0 + r
                    a = _vgather(inv, jnp.full((LANES,), r, jnp.int32))
                    for kk in range(d // LANES):
                        rows_g[row, pl.ds(kk * LANES, LANES)] = (
                            rows_g[row, pl.ds(kk * LANES, LANES)] * a)
                return carry
            lax.fori_loop(0, 4, divg, 0)
            pltpu.sync_copy(rows_g, outn_hbm.at[c, pl.ds(base + t * 64, 64)])

    return k


SPLIT_FRAC = 0.70


def _sc_aggregate(feat, el8, er8, src3, dst3, *, npad, d, nchunk, e_real):
    sp = max(8, int(nchunk * SPLIT_FRAC) // 8 * 8)
    return _sc_aggregate_kernel(npad, d, nchunk, e_real, sp)(
        feat, el8, er8, src3, dst3)


def kernel(inputs, g, W1, al1, ar1, W2, al2, ar2):
    n, f = inputs.shape
    d = W1.shape[1]
    e = g.shape[1]

    npad = -(-n // 2048) * 2048
    nchunk = -(-(-(-e // (NS * 128))) // 16) * 16
    epad = NS * nchunk * 128

    x = jnp.zeros((npad, f), jnp.float32).at[:n, :].set(inputs)
    src = g[0].astype(jnp.int32)
    dst = g[1].astype(jnp.int32)
    src3 = jnp.zeros((epad,), jnp.int32).at[:e].set(src).reshape(NS, nchunk, 128)
    dst3 = jnp.zeros((epad,), jnp.int32).at[:e].set(dst).reshape(NS, nchunk, 128)

    feat1, el81, er81 = _tc_stage(x, None, W1, al1, ar1, npad=npad, d=d)
    pn1 = _sc_aggregate(feat1, el81, er81, src3, dst3,
                        npad=npad, d=d, nchunk=nchunk, e_real=e)
    feat2, el82, er82 = _tc_stage(None, pn1, W2, al2, ar2, npad=npad, d=d)
    pn2 = _sc_aggregate(feat2, el82, er82, src3, dst3,
                        npad=npad, d=d, nchunk=nchunk, e_real=e)
    out = _tc_merge(pn2, npad=npad, d=d)
    return out[:n]


# speedup vs baseline: 1.1895x; 1.0310x over previous
import functools

import jax
import jax.numpy as jnp
from jax import lax
from jax.experimental import pallas as pl
from jax.experimental.pallas import tpu as pltpu
from jax.experimental.pallas import tpu_sc as plsc

NEG_SLOPE = 0.2
EPS = 1e-9

NC = 2
NS = 16
LANES = 16

BM = 1024


def _vgather(x, idx):
    dn = lax.GatherDimensionNumbers(offset_dims=(), collapsed_slice_dims=(0,),
                                    start_index_map=(0,))
    return lax.gather(x, idx[:, None], dn, (1,),
                      mode=lax.GatherScatterMode.PROMISE_IN_BOUNDS)


def _tc_stage(x, pn, W, al, ar, *, npad, d):
    first = x is not None
    grid = npad // BM

    def body(*refs):
        if first:
            x_ref, w_ref, al_ref, ar_ref, feat_ref, el_ref, er_ref = refs
            xb = x_ref[...]
        else:
            pn_ref, w_ref, al_ref, ar_ref, feat_ref, el_ref, er_ref = refs
            xb = pn_ref[0] + pn_ref[1]
        f = jnp.dot(xb, w_ref[...], preferred_element_type=jnp.float32)
        feat_ref[...] = f
        el = jnp.sum(f * al_ref[...], axis=1)
        er = jnp.sum(f * ar_ref[...], axis=1)
        el_ref[...] = jnp.broadcast_to(el[None, :], (8, BM))
        er_ref[...] = jnp.broadcast_to(er[None, :], (8, BM))

    if first:
        data_specs = [pl.BlockSpec((BM, d), lambda i: (i, 0))]
        data_args = (x,)
    else:
        data_specs = [pl.BlockSpec((NC, BM, d), lambda i: (0, i, 0))]
        data_args = (pn,)

    return pl.pallas_call(
        body,
        grid=(grid,),
        in_specs=data_specs + [
            pl.BlockSpec((d, d), lambda i: (0, 0)),
            pl.BlockSpec((1, d), lambda i: (0, 0)),
            pl.BlockSpec((1, d), lambda i: (0, 0)),
        ],
        out_specs=[
            pl.BlockSpec((BM, d), lambda i: (i, 0)),
            pl.BlockSpec((8, BM), lambda i: (0, i)),
            pl.BlockSpec((8, BM), lambda i: (0, i)),
        ],
        out_shape=[
            jax.ShapeDtypeStruct((npad, d), jnp.float32),
            jax.ShapeDtypeStruct((8, npad), jnp.float32),
            jax.ShapeDtypeStruct((8, npad), jnp.float32),
        ],
    )(*data_args, W, al, ar)


def _tc_merge(pn, *, npad, d):
    grid = npad // BM

    def body(pn_ref, out_ref):
        out_ref[...] = pn_ref[0] + pn_ref[1]

    return pl.pallas_call(
        body,
        grid=(grid,),
        in_specs=[pl.BlockSpec((NC, BM, d), lambda i: (0, i, 0))],
        out_specs=pl.BlockSpec((BM, d), lambda i: (i, 0)),
        out_shape=jax.ShapeDtypeStruct((npad, d), jnp.float32),
    )(pn)


@functools.lru_cache(maxsize=None)
def _sc_aggregate_kernel(npad, d, nchunk, e_real, sp):
    eptile = nchunk * 128
    rpt = npad // NS
    nslab = rpt // 64
    mesh = plsc.VectorSubcoreMesh(core_axis_name="c", subcore_axis_name="s")

    @functools.partial(
        pl.kernel,
        out_type=jax.ShapeDtypeStruct((NC, npad, d), jnp.float32),
        mesh=mesh,
        compiler_params=pltpu.CompilerParams(needs_layout_passes=False),
        scratch_types=(
            pltpu.VMEM((npad,), jnp.float32),
            pltpu.VMEM((npad,), jnp.float32),
            pltpu.VMEM((64, d), jnp.float32),
            pltpu.VMEM((64, d), jnp.float32),
            pltpu.VMEM((8, 128), jnp.int32),
            pltpu.VMEM((8, 128), jnp.int32),
            pltpu.VMEM((128,), jnp.int32),
            pltpu.VMEM((128,), jnp.float32),
            pltpu.VMEM((8, 128), jnp.int32),
            pltpu.VMEM((8, 128), jnp.float32),
            pltpu.VMEM((rpt,), jnp.float32),
            pltpu.VMEM_SHARED((npad, d), jnp.float32),
            pltpu.VMEM_SHARED((npad,), jnp.float32),
            pltpu.SemaphoreType.DMA,
            pltpu.SemaphoreType.DMA,
            pltpu.SemaphoreType.DMA,
            pltpu.SemaphoreType.DMA,
            pltpu.SemaphoreType.DMA,
        ),
    )
    def k(feat_hbm, el8_hbm, er8_hbm, src_hbm, dst_hbm, outn_hbm,
          el_v, er_v, rows_g, rows_s, srcst, dstst, ddst_v, dval_v,
          ddst8, dval8, den_v, accn, accd, sem_g, sem_g2, sem_s, sem_s2,
          sem_d):
        c = lax.axis_index("c")
        s = lax.axis_index("s")
        iota16 = lax.iota(jnp.int32, LANES)
        zeros16 = jnp.zeros((LANES,), jnp.float32)
        base = s * rpt

        pltpu.sync_copy(el8_hbm.at[0], el_v)
        pltpu.sync_copy(er8_hbm.at[0], er_v)

        def zrow(r, carry):
            for kk in range(d // LANES):
                rows_g[r, pl.ds(kk * LANES, LANES)] = zeros16
            return carry
        lax.fori_loop(0, 64, zrow, 0)

        def zden(r, carry):
            den_v[pl.ds(r * LANES, LANES)] = zeros16
            return carry
        lax.fori_loop(0, rpt // LANES, zden, 0)

        for t in range(nslab):
            pltpu.sync_copy(rows_g, accn.at[pl.ds(base + t * 64, 64)])
        pltpu.sync_copy(den_v, accd.at[pl.ds(base, rpt)])
        plsc.subcore_barrier()

        lo1 = jnp.where(c == 0, sp, 0)
        ng1 = jnp.where(c == 0, (nchunk - sp) // 8, sp // 8)
        lo2 = jnp.where(c == 0, 0, sp)
        ng2 = jnp.where(c == 0, sp // 8, (nchunk - sp) // 8)

        def wonly16(q, ch, off, j):
            s16 = srcst[q, pl.ds(off + j * LANES, LANES)]
            d16 = dstst[q, pl.ds(off + j * LANES, LANES)]
            ev = plsc.load_gather(el_v, [s16]) + plsc.load_gather(er_v, [d16])
            ev = jnp.where(ev >= 0.0, ev, NEG_SLOPE * ev)
            wv = jnp.exp(ev)
            gid = s * eptile + ch * 128 + off + j * LANES + iota16
            wv = jnp.where(gid < e_real, wv, 0.0)
            return d16, wv

        def wgroup16(q, ch, off, j):
            d16, wv = wonly16(q, ch, off, j)
            ddst_v[pl.ds(off + j * LANES, LANES)] = d16
            dval_v[pl.ds(off + j * LANES, LANES)] = wv
            return wv

        def grp1(g, carry):
            glo = lo1 + g * 8
            pltpu.sync_copy(src_hbm.at[s, pl.ds(glo, 8)], srcst)
            pltpu.sync_copy(dst_hbm.at[s, pl.ds(glo, 8)], dstst)
            for q in range(8):
                ch = glo + q
                for j in range(8):
                    d16, wv = wonly16(q, ch, 0, j)
                    ddst8[q, pl.ds(j * LANES, LANES)] = d16
                    dval8[q, pl.ds(j * LANES, LANES)] = wv
                pltpu.async_copy(dval8.at[q], accd.at[ddst8.at[q]], sem_d, add=True)
            for q in range(8):
                pltpu.make_async_copy(
                    dval8.at[q], accd.at[ddst8.at[q]], sem_d).wait()
            return carry
        lax.fori_loop(0, ng1, grp1, 0)

        def grp2(g, carry):
            glo = lo2 + g * 8
            pltpu.sync_copy(src_hbm.at[s, pl.ds(glo, 8)], srcst)
            pltpu.sync_copy(dst_hbm.at[s, pl.ds(glo, 8)], dstst)

            def scale64(buf, wvs):
                def scale(jj, carry3, _wvs=wvs, _buf=buf):
                    wv = _wvs[0]
                    for jw in range(1, 4):
                        wv = jnp.where(jj == jw, _wvs[jw], wv)
                    for r in range(LANES):
                        row = jj * LANES + r
                        a = _vgather(wv, jnp.full((LANES,), r, jnp.int32))
                        for kk in range(d // LANES):
                            _buf[row, pl.ds(kk * LANES, LANES)] = (
                                _buf[row, pl.ds(kk * LANES, LANES)] * a)
                    return carry3
                lax.fori_loop(0, 4, scale, 0)

            def ch2(q, carry2):
                ch = glo + q
                pltpu.async_copy(
                    feat_hbm.at[srcst.at[q, pl.ds(0, 64)]], rows_g, sem_g)
                pltpu.async_copy(
                    feat_hbm.at[srcst.at[q, pl.ds(64, 64)]], rows_s, sem_g2)
                wvs_a = [wgroup16(q, ch, 0, j) for j in range(4)]
                wvs_b = [wgroup16(q, ch, 64, j) for j in range(4)]
                pltpu.async_copy(dval_v, accd.at[ddst_v], sem_d, add=True)
                pltpu.make_async_copy(
                    feat_hbm.at[srcst.at[q, pl.ds(0, 64)]], rows_g,
                    sem_g).wait()
                scale64(rows_g, wvs_a)
                pltpu.async_copy(
                    rows_g, accn.at[dstst.at[q, pl.ds(0, 64)]], sem_s,
                    add=True)
                pltpu.make_async_copy(
                    feat_hbm.at[srcst.at[q, pl.ds(64, 64)]], rows_s,
                    sem_g2).wait()
                scale64(rows_s, wvs_b)
                pltpu.async_copy(
                    rows_s, accn.at[dstst.at[q, pl.ds(64, 64)]], sem_s2,
                    add=True)
                pltpu.make_async_copy(
                    rows_g, accn.at[dstst.at[q, pl.ds(0, 64)]], sem_s).wait()
                pltpu.make_async_copy(dval_v, accd.at[ddst_v], sem_d).wait()
                pltpu.make_async_copy(
                    rows_s, accn.at[dstst.at[q, pl.ds(64, 64)]],
                    sem_s2).wait()
                return carry2
            lax.fori_loop(0, 8, ch2, 0)
            return carry
        lax.fori_loop(0, ng2, grp2, 0)
        plsc.subcore_barrier()

        pltpu.sync_copy(accd.at[pl.ds(base, rpt)], den_v)
        for t in range(nslab):
            pltpu.sync_copy(accn.at[pl.ds(base + t * 64, 64)], rows_g)

            def divg(jj, carry, _t=t):
                r0 = jj * LANES
                dv = den_v[pl.ds(_t * 64 + r0, LANES)]
                inv = 1.0 / (dv + EPS)
                for r in range(LANES):
                    row = r0 + r
                    a = _vgather(inv, jnp.full((LANES,), r, jnp.int32))
                    for kk in range(d // LANES):
                        rows_g[row, pl.ds(kk * LANES, LANES)] = (
                            rows_g[row, pl.ds(kk * LANES, LANES)] * a)
                return carry
            lax.fori_loop(0, 4, divg, 0)
            pltpu.sync_copy(rows_g, outn_hbm.at[c, pl.ds(base + t * 64, 64)])

    return k


SPLIT_FRAC = 0.70


def _sc_aggregate(feat, el8, er8, src3, dst3, *, npad, d, nchunk, e_real):
    sp = max(8, int(nchunk * SPLIT_FRAC) // 8 * 8)
    return _sc_aggregate_kernel(npad, d, nchunk, e_real, sp)(
        feat, el8, er8, src3, dst3)


def kernel(inputs, g, W1, al1, ar1, W2, al2, ar2):
    n, f = inputs.shape
    d = W1.shape[1]
    e = g.shape[1]

    npad = -(-n // 2048) * 2048
    nchunk = -(-(-(-e // (NS * 128))) // 16) * 16
    epad = NS * nchunk * 128

    x = jnp.zeros((npad, f), jnp.float32).at[:n, :].set(inputs)
    src = g[0].astype(jnp.int32)
    dst = g[1].astype(jnp.int32)
    src3 = jnp.zeros((epad,), jnp.int32).at[:e].set(src).reshape(NS, nchunk, 128)
    dst3 = jnp.zeros((epad,), jnp.int32).at[:e].set(dst).reshape(NS, nchunk, 128)

    feat1, el81, er81 = _tc_stage(x, None, W1, al1, ar1, npad=npad, d=d)
    pn1 = _sc_aggregate(feat1, el81, er81, src3, dst3,
                        npad=npad, d=d, nchunk=nchunk, e_real=e)
    feat2, el82, er82 = _tc_stage(None, pn1, W2, al2, ar2, npad=npad, d=d)
    pn2 = _sc_aggregate(feat2, el82, er82, src3, dst3,
                        npad=npad, d=d, nchunk=nchunk, e_real=e)
    out = _tc_merge(pn2, npad=npad, d=d)
    return out[:n]
